# Initial kernel scaffold; baseline (speedup 1.0000x reference)
import jax
import jax.numpy as jnp
from jax.experimental import pallas as pl


def kernel(x, edge_index, edge_type, w1, root1, b1, w2, root2, b2):
    raise NotImplementedError("write your pallas kernel here")


# speedup vs baseline: 1.4883x; 1.4883x over previous
import functools

import jax
import jax.numpy as jnp
from jax import lax
from jax.experimental import pallas as pl
from jax.experimental.pallas import tpu as pltpu
from jax.experimental.pallas import tpu_sc as plsc

N = 10000
E = 160000
NF = 256
HC = 256
NC = 128
R = 4

NSC = 2
NTILE = 16
LANES = 16
D = 128
HALF = N // NSC
ROWS_PT = 320
ACC_ROWS = NTILE * ROWS_PT
TRASH = HALF
BASEPAD = NSC * ACC_ROWS
EP_TILE = E // NTILE
CHUNK = 80
NCHUNK = EP_TILE // CHUNK


def _mesh():
    return plsc.VectorSubcoreMesh(
        core_axis_name="c", subcore_axis_name="s", num_cores=NSC,
        num_subcores=NTILE)


@functools.cache
def _make_count_kernel():
    @functools.partial(
        pl.kernel,
        out_type=jax.ShapeDtypeStruct((NSC, ACC_ROWS, D), jnp.float32),
        mesh=_mesh(),
        scratch_types=[
            pltpu.VMEM((EP_TILE,), jnp.int32),
            pltpu.VMEM((EP_TILE,), jnp.int32),
            pltpu.VMEM((CHUNK, D), jnp.float32),
            pltpu.VMEM((CHUNK,), jnp.int32),
            pltpu.VMEM((CHUNK,), jnp.int32),
            pltpu.VMEM_SHARED((ACC_ROWS, D), jnp.float32),
            pltpu.SemaphoreType.DMA,
        ],
    )
    def _cnt(oh_hbm, zeros_hbm, dst_hbm, typ_hbm, out_hbm,
             dst_v, typ_v, rows_v, gidx_v, sidx_v, acc_sh, sem):
        c = lax.axis_index("c")
        t = lax.axis_index("s")
        lo = c * HALF
        r0 = t * ROWS_PT

        pltpu.sync_copy(zeros_hbm.at[pl.ds(r0, ROWS_PT)],
                        acc_sh.at[pl.ds(r0, ROWS_PT)])
        e0 = t * EP_TILE
        pltpu.sync_copy(dst_hbm.at[pl.ds(e0, EP_TILE)], dst_v)
        pltpu.sync_copy(typ_hbm.at[pl.ds(e0, EP_TILE)], typ_v)
        plsc.subcore_barrier()

        def chunk_body(k, carry):
            for j in range(CHUNK // LANES):
                sl = pl.ds(j * LANES, LANES)
                esl = pl.ds(k * CHUNK + j * LANES, LANES)
                tv = typ_v[esl]
                dv = dst_v[esl]
                gidx_v[sl] = tv
                li = dv - lo
                ok = (li >= 0) & (li < HALF)
                sidx_v[sl] = jnp.where(ok, li, TRASH)
            pltpu.async_copy(oh_hbm.at[gidx_v], rows_v, sem).wait()
            pltpu.sync_copy(rows_v, acc_sh.at[sidx_v], add=True)
            return carry
        lax.fori_loop(0, NCHUNK, chunk_body, 0)

        plsc.subcore_barrier()
        pltpu.sync_copy(acc_sh.at[pl.ds(r0, ROWS_PT)],
                        out_hbm.at[c, pl.ds(r0, ROWS_PT)])
    return _cnt


@functools.cache
def _make_msg_kernel(npass):
    @functools.partial(
        pl.kernel,
        out_type=jax.ShapeDtypeStruct((npass, NSC, ACC_ROWS, D),
                                      jnp.float32),
        mesh=_mesh(),
        scratch_types=[
            pltpu.VMEM((EP_TILE,), jnp.int32),
            pltpu.VMEM((EP_TILE,), jnp.int32),
            pltpu.VMEM((EP_TILE,), jnp.int32),
            pltpu.VMEM((CHUNK, D), jnp.float32),
            pltpu.VMEM((CHUNK, D), jnp.float32),
            pltpu.VMEM((CHUNK,), jnp.int32),
            pltpu.VMEM((CHUNK,), jnp.int32),
            pltpu.VMEM((CHUNK,), jnp.int32),
            pltpu.VMEM_SHARED((ACC_ROWS, D), jnp.float32),
            pltpu.SemaphoreType.DMA,
        ],
    )
    def _msg(*args):
        bases = args[:npass]
        (hmsg_hbm, cnt_hbm, src_hbm, dst_hbm, typ_hbm, out_hbm,
         src_v, dst_v, typ_v, rows_v, crow_v, gidx_v, cidx_v, sidx_v,
         acc_sh, sem) = args[npass:]
        c = lax.axis_index("c")
        t = lax.axis_index("s")
        lo = c * HALF
        r0 = t * ROWS_PT
        ones16 = jnp.ones((LANES,), jnp.float32)

        e0 = t * EP_TILE
        pltpu.sync_copy(src_hbm.at[pl.ds(e0, EP_TILE)], src_v)
        pltpu.sync_copy(dst_hbm.at[pl.ds(e0, EP_TILE)], dst_v)
        pltpu.sync_copy(typ_hbm.at[pl.ds(e0, EP_TILE)], typ_v)

        for f in range(npass):
            base_hbm = bases[f]
            pltpu.sync_copy(base_hbm.at[pl.ds(c * ACC_ROWS + r0, ROWS_PT)],
                            acc_sh.at[pl.ds(r0, ROWS_PT)])
            plsc.subcore_barrier()

            def chunk_body(k, carry):
                for j in range(CHUNK // LANES):
                    sl = pl.ds(j * LANES, LANES)
                    esl = pl.ds(k * CHUNK + j * LANES, LANES)
                    sv = src_v[esl]
                    tv = typ_v[esl]
                    dv = dst_v[esl]
                    gidx_v[sl] = (sv * R + tv) * npass + f
                    cidx_v[sl] = tv * N + dv
                    li = dv - lo
                    ok = (li >= 0) & (li < HALF)
                    sidx_v[sl] = jnp.where(ok, li, TRASH)
                d1 = pltpu.async_copy(hmsg_hbm.at[gidx_v], rows_v, sem)
                d2 = pltpu.async_copy(cnt_hbm.at[cidx_v], crow_v, sem)
                d1.wait()
                d2.wait()

                def rbody(i, carry2):
                    bc = ones16 / jnp.maximum(
                        crow_v[i, pl.ds(0, LANES)], ones16)
                    for j2 in range(D // LANES):
                        sl2 = pl.ds(j2 * LANES, LANES)
                        rows_v[i, sl2] = rows_v[i, sl2] * bc
                    return carry2
                lax.fori_loop(0, CHUNK, rbody, 0)
                pltpu.sync_copy(rows_v, acc_sh.at[sidx_v], add=True)
                return carry
            lax.fori_loop(0, NCHUNK, chunk_body, 0)

            plsc.subcore_barrier()
            pltpu.sync_copy(acc_sh.at[pl.ds(r0, ROWS_PT)],
                            out_hbm.at[f, c, pl.ds(r0, ROWS_PT)])
            if f + 1 < npass:
                plsc.subcore_barrier()
    return _msg


def _tc_layer(x, root, wcat, b2d, relu, d):
    bm = 1000
    grid = (N // bm,)

    def body(x_ref, root_ref, wcat_ref, b_ref, base_ref, hmsg_ref):
        xb = x_ref[...]
        if relu:
            xb = jnp.maximum(xb, 0.0)
        base_ref[...] = (
            jnp.dot(xb, root_ref[...], preferred_element_type=jnp.float32)
            + b_ref[...])
        hmsg_ref[...] = jnp.dot(
            xb, wcat_ref[...], preferred_element_type=jnp.float32)

    k = x.shape[1]
    return pl.pallas_call(
        body,
        grid=grid,
        in_specs=[
            pl.BlockSpec((bm, k), lambda i: (i, 0)),
            pl.BlockSpec((k, d), lambda i: (0, 0)),
            pl.BlockSpec((k, R * d), lambda i: (0, 0)),
            pl.BlockSpec((1, d), lambda i: (0, 0)),
        ],
        out_specs=[
            pl.BlockSpec((bm, d), lambda i: (i, 0)),
            pl.BlockSpec((bm, R * d), lambda i: (i, 0)),
        ],
        out_shape=[
            jax.ShapeDtypeStruct((N, d), jnp.float32),
            jax.ShapeDtypeStruct((N, R * d), jnp.float32),
        ],
    )(x, root, wcat, b2d)


def _halves(p):
    return jnp.concatenate([p[0, :HALF], p[1, :HALF]], axis=0)


def _pad_rows(a):
    return jnp.concatenate(
        [jnp.pad(a[:HALF], ((0, ACC_ROWS - HALF), (0, 0))),
         jnp.pad(a[HALF:], ((0, ACC_ROWS - HALF), (0, 0)))], axis=0)


def kernel(x, edge_index, edge_type, w1, root1, b1, w2, root2, b2):
    src = edge_index[0]
    dst = edge_index[1]
    typ = edge_type

    oh_tab = jnp.pad(jnp.repeat(jnp.eye(R, dtype=jnp.float32), LANES, axis=1),
                     ((0, 0), (0, D - R * LANES)))
    zeros = jnp.zeros((ACC_ROWS, D), jnp.float32)
    cntq = _make_count_kernel()(oh_tab, zeros, dst, typ)
    cvals = _halves(cntq)[:, 0:R * LANES:LANES]
    cnt = jnp.broadcast_to(
        cvals.T.reshape(R * N, 1), (R * N, D))

    wcat1 = jnp.transpose(w1, (1, 0, 2)).reshape(NF, R * HC)
    wcat2 = jnp.transpose(w2, (1, 0, 2)).reshape(HC, R * NC)

    base1, hmsg1 = _tc_layer(x, root1, wcat1, b1.reshape(1, HC), False, HC)
    h1v = hmsg1.reshape(N * R * 2, D)
    embp = _make_msg_kernel(2)(
        _pad_rows(base1[:, :D]), _pad_rows(base1[:, D:]), h1v,
        cnt, src, dst, typ)
    emb = jnp.concatenate([_halves(embp[0]), _halves(embp[1])], axis=1)

    base2, hmsg2 = _tc_layer(emb, root2, wcat2, b2.reshape(1, NC), True, NC)
    logp = _make_msg_kernel(1)(
        _pad_rows(base2), hmsg2.reshape(N * R, NC), cnt, src, dst, typ)
    logits = _halves(logp[0])

    return (logits, emb)


# speedup vs baseline: 1.6253x; 1.0920x over previous
import functools

import jax
import jax.numpy as jnp
from jax import lax
from jax.experimental import pallas as pl
from jax.experimental.pallas import tpu as pltpu
from jax.experimental.pallas import tpu_sc as plsc

N = 10000
E = 160000
NF = 256
HC = 256
NC = 128
R = 4

NSC = 2
NTILE = 16
LANES = 16
D = 128
HALF = N // NSC
ROWS_PT = 320
ACC_ROWS = NTILE * ROWS_PT
TRASH = HALF
BASEPAD = NSC * ACC_ROWS
EP_TILE = E // NTILE
CHUNK = 80
NCHUNK = EP_TILE // CHUNK


def _mesh():
    return plsc.VectorSubcoreMesh(
        core_axis_name="c", subcore_axis_name="s", num_cores=NSC,
        num_subcores=NTILE)


@functools.cache
def _make_count_kernel():
    @functools.partial(
        pl.kernel,
        out_type=jax.ShapeDtypeStruct((NSC, ACC_ROWS, D), jnp.float32),
        mesh=_mesh(),
        scratch_types=[
            pltpu.VMEM((EP_TILE,), jnp.int32),
            pltpu.VMEM((EP_TILE,), jnp.int32),
            pltpu.VMEM((CHUNK, D), jnp.float32),
            pltpu.VMEM((CHUNK,), jnp.int32),
            pltpu.VMEM((CHUNK,), jnp.int32),
            pltpu.VMEM_SHARED((ACC_ROWS, D), jnp.float32),
            pltpu.SemaphoreType.DMA,
        ],
    )
    def _cnt(oh_hbm, zeros_hbm, dst_hbm, typ_hbm, out_hbm,
             dst_v, typ_v, rows_v, gidx_v, sidx_v, acc_sh, sem):
        c = lax.axis_index("c")
        t = lax.axis_index("s")
        lo = c * HALF
        r0 = t * ROWS_PT

        pltpu.sync_copy(zeros_hbm.at[pl.ds(r0, ROWS_PT)],
                        acc_sh.at[pl.ds(r0, ROWS_PT)])
        e0 = t * EP_TILE
        pltpu.sync_copy(dst_hbm.at[pl.ds(e0, EP_TILE)], dst_v)
        pltpu.sync_copy(typ_hbm.at[pl.ds(e0, EP_TILE)], typ_v)
        plsc.subcore_barrier()

        def chunk_body(k, carry):
            for j in range(CHUNK // LANES):
                sl = pl.ds(j * LANES, LANES)
                esl = pl.ds(k * CHUNK + j * LANES, LANES)
                tv = typ_v[esl]
                dv = dst_v[esl]
                gidx_v[sl] = tv
                li = dv - lo
                ok = (li >= 0) & (li < HALF)
                sidx_v[sl] = jnp.where(ok, li, TRASH)
            pltpu.async_copy(oh_hbm.at[gidx_v], rows_v, sem).wait()
            pltpu.sync_copy(rows_v, acc_sh.at[sidx_v], add=True)
            return carry
        lax.fori_loop(0, NCHUNK, chunk_body, 0)

        plsc.subcore_barrier()
        pltpu.sync_copy(acc_sh.at[pl.ds(r0, ROWS_PT)],
                        out_hbm.at[c, pl.ds(r0, ROWS_PT)])
    return _cnt


@functools.cache
def _make_msg_kernel(npass):
    @functools.partial(
        pl.kernel,
        out_type=jax.ShapeDtypeStruct((npass, NSC, ACC_ROWS, D),
                                      jnp.float32),
        mesh=_mesh(),
        scratch_types=[
            pltpu.VMEM((EP_TILE,), jnp.int32),
            pltpu.VMEM((EP_TILE,), jnp.int32),
            pltpu.VMEM((EP_TILE,), jnp.int32),
            pltpu.VMEM((2, CHUNK, D), jnp.float32),
            pltpu.VMEM((2, CHUNK, D), jnp.float32),
            pltpu.VMEM((2, CHUNK), jnp.int32),
            pltpu.VMEM((2, CHUNK), jnp.int32),
            pltpu.VMEM((2, CHUNK), jnp.int32),
            pltpu.VMEM_SHARED((ACC_ROWS, D), jnp.float32),
            pltpu.SemaphoreType.DMA,
            pltpu.SemaphoreType.DMA,
            pltpu.SemaphoreType.DMA,
            pltpu.SemaphoreType.DMA,
            pltpu.SemaphoreType.DMA,
            pltpu.SemaphoreType.DMA,
        ],
    )
    def _msg(*args):
        bases = args[:npass]
        (hmsg_hbm, cnt_hbm, src_hbm, dst_hbm, typ_hbm, out_hbm,
         src_v, dst_v, typ_v, rows_v, crow_v, gidx_v, cidx_v, sidx_v,
         acc_sh, sgm0, sgm1, sgc0, sgc1, ss0, ss1) = args[npass:]
        c = lax.axis_index("c")
        t = lax.axis_index("s")
        lo = c * HALF
        r0 = t * ROWS_PT
        sgm = (sgm0, sgm1)
        sgc = (sgc0, sgc1)
        ss = (ss0, ss1)

        e0 = t * EP_TILE
        pltpu.sync_copy(src_hbm.at[pl.ds(e0, EP_TILE)], src_v)
        pltpu.sync_copy(dst_hbm.at[pl.ds(e0, EP_TILE)], dst_v)
        pltpu.sync_copy(typ_hbm.at[pl.ds(e0, EP_TILE)], typ_v)

        for f in range(npass):
            base_hbm = bases[f]
            pltpu.sync_copy(base_hbm.at[pl.ds(c * ACC_ROWS + r0, ROWS_PT)],
                            acc_sh.at[pl.ds(r0, ROWS_PT)])
            plsc.subcore_barrier()

            def calc_idx(k, b):
                for j in range(CHUNK // LANES):
                    sl = pl.ds(j * LANES, LANES)
                    esl = pl.ds(k * CHUNK + j * LANES, LANES)
                    sv = src_v[esl]
                    tv = typ_v[esl]
                    dv = dst_v[esl]
                    gidx_v[b, sl] = (sv * R + tv) * npass + f
                    cidx_v[b, sl] = tv * N + dv
                    li = dv - lo
                    ok = (li >= 0) & (li < HALF)
                    sidx_v[b, sl] = jnp.where(ok, li, TRASH)

            def fire_gathers(b):
                pltpu.async_copy(hmsg_hbm.at[gidx_v.at[b]], rows_v.at[b],
                                 sgm[b])
                pltpu.async_copy(cnt_hbm.at[cidx_v.at[b]], crow_v.at[b],
                                 sgc[b])

            def wait_gathers(b):
                pltpu.make_async_copy(hmsg_hbm.at[gidx_v.at[b]],
                                      rows_v.at[b], sgm[b]).wait()
                pltpu.make_async_copy(cnt_hbm.at[cidx_v.at[b]],
                                      crow_v.at[b], sgc[b]).wait()

            def wait_scatter(b):
                pltpu.make_async_copy(rows_v.at[b],
                                      acc_sh.at[sidx_v.at[b]],
                                      ss[b]).wait()

            def scale(b):
                def rbody(i, carry2):
                    bc = crow_v[b, i, pl.ds(0, LANES)]
                    for j2 in range(D // LANES):
                        sl2 = pl.ds(j2 * LANES, LANES)
                        rows_v[b, i, sl2] = rows_v[b, i, sl2] * bc
                    return carry2
                lax.fori_loop(0, CHUNK, rbody, 0)

            def step(k, b, prefetch):
                wait_gathers(b)
                scale(b)
                pltpu.async_copy(rows_v.at[b], acc_sh.at[sidx_v.at[b]],
                                 ss[b], add=True)
                wait_scatter(b)
                if prefetch:
                    @pl.when(k + 2 < NCHUNK)
                    def _():
                        calc_idx(k + 2, b)
                        fire_gathers(b)

            for b in range(2):
                calc_idx(b, b)
                fire_gathers(b)

            def pair_body(k2, carry):
                for b in range(2):
                    step(k2 * 2 + b, b, True)
                return carry
            lax.fori_loop(0, NCHUNK // 2, pair_body, 0)
            step(NCHUNK - 1, (NCHUNK - 1) % 2, False)

            plsc.subcore_barrier()
            pltpu.sync_copy(acc_sh.at[pl.ds(r0, ROWS_PT)],
                            out_hbm.at[f, c, pl.ds(r0, ROWS_PT)])
            if f + 1 < npass:
                plsc.subcore_barrier()
    return _msg


def _tc_layer(x, root, wcat, b2d, relu, d):
    bm = 1000
    grid = (N // bm,)

    def body(x_ref, root_ref, wcat_ref, b_ref, base_ref, hmsg_ref):
        xb = x_ref[...]
        if relu:
            xb = jnp.maximum(xb, 0.0)
        base_ref[...] = (
            jnp.dot(xb, root_ref[...], preferred_element_type=jnp.float32)
            + b_ref[...])
        hmsg_ref[...] = jnp.dot(
            xb, wcat_ref[...], preferred_element_type=jnp.float32)

    k = x.shape[1]
    return pl.pallas_call(
        body,
        grid=grid,
        in_specs=[
            pl.BlockSpec((bm, k), lambda i: (i, 0)),
            pl.BlockSpec((k, d), lambda i: (0, 0)),
            pl.BlockSpec((k, R * d), lambda i: (0, 0)),
            pl.BlockSpec((1, d), lambda i: (0, 0)),
        ],
        out_specs=[
            pl.BlockSpec((bm, d), lambda i: (i, 0)),
            pl.BlockSpec((bm, R * d), lambda i: (i, 0)),
        ],
        out_shape=[
            jax.ShapeDtypeStruct((N, d), jnp.float32),
            jax.ShapeDtypeStruct((N, R * d), jnp.float32),
        ],
    )(x, root, wcat, b2d)


def _halves(p):
    return jnp.concatenate([p[0, :HALF], p[1, :HALF]], axis=0)


def _pad_rows(a):
    return jnp.concatenate(
        [jnp.pad(a[:HALF], ((0, ACC_ROWS - HALF), (0, 0))),
         jnp.pad(a[HALF:], ((0, ACC_ROWS - HALF), (0, 0)))], axis=0)


def kernel(x, edge_index, edge_type, w1, root1, b1, w2, root2, b2):
    src = edge_index[0]
    dst = edge_index[1]
    typ = edge_type

    oh_tab = jnp.pad(jnp.repeat(jnp.eye(R, dtype=jnp.float32), LANES, axis=1),
                     ((0, 0), (0, D - R * LANES)))
    zeros = jnp.zeros((ACC_ROWS, D), jnp.float32)
    cntq = _make_count_kernel()(oh_tab, zeros, dst, typ)
    cvals = _halves(cntq)[:, 0:R * LANES:LANES]
    scal = 1.0 / jnp.maximum(cvals, 1.0)
    cnt = jnp.broadcast_to(
        scal.T.reshape(R * N, 1), (R * N, D))

    wcat1 = jnp.transpose(w1, (1, 0, 2)).reshape(NF, R * HC)
    wcat2 = jnp.transpose(w2, (1, 0, 2)).reshape(HC, R * NC)

    base1, hmsg1 = _tc_layer(x, root1, wcat1, b1.reshape(1, HC), False, HC)
    h1v = hmsg1.reshape(N * R * 2, D)
    embp = _make_msg_kernel(2)(
        _pad_rows(base1[:, :D]), _pad_rows(base1[:, D:]), h1v,
        cnt, src, dst, typ)
    emb = jnp.concatenate([_halves(embp[0]), _halves(embp[1])], axis=1)

    base2, hmsg2 = _tc_layer(emb, root2, wcat2, b2.reshape(1, NC), True, NC)
    logp = _make_msg_kernel(1)(
        _pad_rows(base2), hmsg2.reshape(N * R, NC), cnt, src, dst, typ)
    logits = _halves(logp[0])

    return (logits, emb)


# speedup vs baseline: 6.5588x; 4.0355x over previous
import functools

import jax
import jax.numpy as jnp
from jax import lax
from jax.experimental import pallas as pl
from jax.experimental.pallas import tpu as pltpu
from jax.experimental.pallas import tpu_sc as plsc

N = 10000
E = 160000
NF = 256
HC = 256
NC = 128
R = 4

NSC = 2
NTILE = 16
LANES = 16
D = 128
HALF = N // NSC
ROWS_PT = 320
ACC_ROWS = NTILE * ROWS_PT
TRASH = HALF
BASEPAD = NSC * ACC_ROWS
EP_TILE = E // NTILE
CHUNK = 80
NCHUNK = EP_TILE // CHUNK
OH_REP = 1024


def _mesh():
    return plsc.VectorSubcoreMesh(
        core_axis_name="c", subcore_axis_name="s", num_cores=NSC,
        num_subcores=NTILE)


@functools.cache
def _make_count_kernel():
    @functools.partial(
        pl.kernel,
        out_type=jax.ShapeDtypeStruct((NSC, ACC_ROWS, D), jnp.float32),
        mesh=_mesh(),
        scratch_types=[
            pltpu.VMEM((EP_TILE,), jnp.int32),
            pltpu.VMEM((EP_TILE,), jnp.int32),
            pltpu.VMEM((CHUNK, D), jnp.float32),
            pltpu.VMEM((CHUNK,), jnp.int32),
            pltpu.VMEM((CHUNK,), jnp.int32),
            pltpu.VMEM_SHARED((ACC_ROWS, D), jnp.float32),
            pltpu.SemaphoreType.DMA,
        ],
    )
    def _cnt(oh_hbm, zeros_hbm, dst_hbm, typ_hbm, out_hbm,
             dst_v, typ_v, rows_v, gidx_v, sidx_v, acc_sh, sem):
        c = lax.axis_index("c")
        t = lax.axis_index("s")
        lo = c * HALF
        r0 = t * ROWS_PT

        pltpu.sync_copy(zeros_hbm.at[pl.ds(r0, ROWS_PT)],
                        acc_sh.at[pl.ds(r0, ROWS_PT)])
        e0 = t * EP_TILE
        pltpu.sync_copy(dst_hbm.at[pl.ds(e0, EP_TILE)], dst_v)
        pltpu.sync_copy(typ_hbm.at[pl.ds(e0, EP_TILE)], typ_v)
        plsc.subcore_barrier()

        def chunk_body(k, carry):
            for j in range(CHUNK // LANES):
                sl = pl.ds(j * LANES, LANES)
                esl = pl.ds(k * CHUNK + j * LANES, LANES)
                tv = typ_v[esl]
                dv = dst_v[esl]
                gidx_v[sl] = tv * OH_REP + (dv & (OH_REP - 1))
                li = dv - lo
                ok = (li >= 0) & (li < HALF)
                sidx_v[sl] = jnp.where(ok, li, TRASH)
            pltpu.async_copy(oh_hbm.at[gidx_v], rows_v, sem).wait()
            pltpu.sync_copy(rows_v, acc_sh.at[sidx_v], add=True)
            return carry
        lax.fori_loop(0, NCHUNK, chunk_body, 0)

        plsc.subcore_barrier()
        pltpu.sync_copy(acc_sh.at[pl.ds(r0, ROWS_PT)],
                        out_hbm.at[c, pl.ds(r0, ROWS_PT)])
    return _cnt


@functools.cache
def _make_msg_kernel(npass):
    @functools.partial(
        pl.kernel,
        out_type=jax.ShapeDtypeStruct((npass, NSC, ACC_ROWS, D),
                                      jnp.float32),
        mesh=_mesh(),
        scratch_types=[
            pltpu.VMEM((EP_TILE,), jnp.int32),
            pltpu.VMEM((EP_TILE,), jnp.int32),
            pltpu.VMEM((EP_TILE,), jnp.int32),
            pltpu.VMEM((2, CHUNK, D), jnp.float32),
            pltpu.VMEM((2, CHUNK, D), jnp.float32),
            pltpu.VMEM((2, CHUNK), jnp.int32),
            pltpu.VMEM((2, CHUNK), jnp.int32),
            pltpu.VMEM((2, CHUNK), jnp.int32),
            pltpu.VMEM_SHARED((ACC_ROWS, D), jnp.float32),
            pltpu.SemaphoreType.DMA,
            pltpu.SemaphoreType.DMA,
            pltpu.SemaphoreType.DMA,
            pltpu.SemaphoreType.DMA,
            pltpu.SemaphoreType.DMA,
            pltpu.SemaphoreType.DMA,
        ],
    )
    def _msg(*args):
        bases = args[:npass]
        (hmsg_hbm, cnt_hbm, src_hbm, dst_hbm, typ_hbm, out_hbm,
         src_v, dst_v, typ_v, rows_v, crow_v, gidx_v, cidx_v, sidx_v,
         acc_sh, sgm0, sgm1, sgc0, sgc1, ss0, ss1) = args[npass:]
        c = lax.axis_index("c")
        t = lax.axis_index("s")
        lo = c * HALF
        r0 = t * ROWS_PT
        sgm = (sgm0, sgm1)
        sgc = (sgc0, sgc1)
        ss = (ss0, ss1)

        e0 = t * EP_TILE
        pltpu.sync_copy(src_hbm.at[pl.ds(e0, EP_TILE)], src_v)
        pltpu.sync_copy(dst_hbm.at[pl.ds(e0, EP_TILE)], dst_v)
        pltpu.sync_copy(typ_hbm.at[pl.ds(e0, EP_TILE)], typ_v)

        for f in range(npass):
            base_hbm = bases[f]
            pltpu.sync_copy(base_hbm.at[pl.ds(c * ACC_ROWS + r0, ROWS_PT)],
                            acc_sh.at[pl.ds(r0, ROWS_PT)])
            plsc.subcore_barrier()

            def calc_idx(k, b):
                for j in range(CHUNK // LANES):
                    sl = pl.ds(j * LANES, LANES)
                    esl = pl.ds(k * CHUNK + j * LANES, LANES)
                    sv = src_v[esl]
                    tv = typ_v[esl]
                    dv = dst_v[esl]
                    gidx_v[b, sl] = (sv * R + tv) * npass + f
                    cidx_v[b, sl] = tv * N + dv
                    li = dv - lo
                    ok = (li >= 0) & (li < HALF)
                    sidx_v[b, sl] = jnp.where(ok, li, TRASH)

            def fire_gathers(b):
                pltpu.async_copy(hmsg_hbm.at[gidx_v.at[b]], rows_v.at[b],
                                 sgm[b])
                pltpu.async_copy(cnt_hbm.at[cidx_v.at[b]], crow_v.at[b],
                                 sgc[b])

            def wait_gathers(b):
                pltpu.make_async_copy(hmsg_hbm.at[gidx_v.at[b]],
                                      rows_v.at[b], sgm[b]).wait()
                pltpu.make_async_copy(cnt_hbm.at[cidx_v.at[b]],
                                      crow_v.at[b], sgc[b]).wait()

            def wait_scatter(b):
                pltpu.make_async_copy(rows_v.at[b],
                                      acc_sh.at[sidx_v.at[b]],
                                      ss[b]).wait()

            def scale(b):
                def rbody(i, carry2):
                    bc = crow_v[b, i, pl.ds(0, LANES)]
                    for j2 in range(D // LANES):
                        sl2 = pl.ds(j2 * LANES, LANES)
                        rows_v[b, i, sl2] = rows_v[b, i, sl2] * bc
                    return carry2
                lax.fori_loop(0, CHUNK, rbody, 0)

            def step(k, b, prefetch):
                wait_gathers(b)
                scale(b)
                pltpu.async_copy(rows_v.at[b], acc_sh.at[sidx_v.at[b]],
                                 ss[b], add=True)
                wait_scatter(b)
                if prefetch:
                    @pl.when(k + 2 < NCHUNK)
                    def _():
                        calc_idx(k + 2, b)
                        fire_gathers(b)

            for b in range(2):
                calc_idx(b, b)
                fire_gathers(b)

            def pair_body(k2, carry):
                for b in range(2):
                    step(k2 * 2 + b, b, True)
                return carry
            lax.fori_loop(0, NCHUNK // 2, pair_body, 0)
            step(NCHUNK - 1, (NCHUNK - 1) % 2, False)

            plsc.subcore_barrier()
            pltpu.sync_copy(acc_sh.at[pl.ds(r0, ROWS_PT)],
                            out_hbm.at[f, c, pl.ds(r0, ROWS_PT)])
            if f + 1 < npass:
                plsc.subcore_barrier()
    return _msg


def _tc_layer(x, root, wcat, b2d, relu, d):
    bm = 1000
    grid = (N // bm,)

    def body(x_ref, root_ref, wcat_ref, b_ref, base_ref, hmsg_ref):
        xb = x_ref[...]
        if relu:
            xb = jnp.maximum(xb, 0.0)
        base_ref[...] = (
            jnp.dot(xb, root_ref[...], preferred_element_type=jnp.float32)
            + b_ref[...])
        hmsg_ref[...] = jnp.dot(
            xb, wcat_ref[...], preferred_element_type=jnp.float32)

    k = x.shape[1]
    return pl.pallas_call(
        body,
        grid=grid,
        in_specs=[
            pl.BlockSpec((bm, k), lambda i: (i, 0)),
            pl.BlockSpec((k, d), lambda i: (0, 0)),
            pl.BlockSpec((k, R * d), lambda i: (0, 0)),
            pl.BlockSpec((1, d), lambda i: (0, 0)),
        ],
        out_specs=[
            pl.BlockSpec((bm, d), lambda i: (i, 0)),
            pl.BlockSpec((bm, R * d), lambda i: (i, 0)),
        ],
        out_shape=[
            jax.ShapeDtypeStruct((N, d), jnp.float32),
            jax.ShapeDtypeStruct((N, R * d), jnp.float32),
        ],
    )(x, root, wcat, b2d)


def _halves(p):
    return jnp.concatenate([p[0, :HALF], p[1, :HALF]], axis=0)


def _pad_rows(a):
    return jnp.concatenate(
        [jnp.pad(a[:HALF], ((0, ACC_ROWS - HALF), (0, 0))),
         jnp.pad(a[HALF:], ((0, ACC_ROWS - HALF), (0, 0)))], axis=0)


def kernel(x, edge_index, edge_type, w1, root1, b1, w2, root2, b2):
    src = edge_index[0]
    dst = edge_index[1]
    typ = edge_type

    oh1 = jnp.pad(jnp.repeat(jnp.eye(R, dtype=jnp.float32), LANES, axis=1),
                  ((0, 0), (0, D - R * LANES)))
    oh_tab = jnp.broadcast_to(
        oh1[:, None, :], (R, OH_REP, D)).reshape(R * OH_REP, D)
    zeros = jnp.zeros((ACC_ROWS, D), jnp.float32)
    cntq = _make_count_kernel()(oh_tab, zeros, dst, typ)
    cvals = _halves(cntq)[:, 0:R * LANES:LANES]
    scal = 1.0 / jnp.maximum(cvals, 1.0)
    cnt = jnp.broadcast_to(
        scal.T.reshape(R * N, 1), (R * N, D))

    wcat1 = jnp.transpose(w1, (1, 0, 2)).reshape(NF, R * HC)
    wcat2 = jnp.transpose(w2, (1, 0, 2)).reshape(HC, R * NC)

    base1, hmsg1 = _tc_layer(x, root1, wcat1, b1.reshape(1, HC), False, HC)
    h1v = hmsg1.reshape(N * R * 2, D)
    embp = _make_msg_kernel(2)(
        _pad_rows(base1[:, :D]), _pad_rows(base1[:, D:]), h1v,
        cnt, src, dst, typ)
    emb = jnp.concatenate([_halves(embp[0]), _halves(embp[1])], axis=1)

    base2, hmsg2 = _tc_layer(emb, root2, wcat2, b2.reshape(1, NC), True, NC)
    logp = _make_msg_kernel(1)(
        _pad_rows(base2), hmsg2.reshape(N * R, NC), cnt, src, dst, typ)
    logits = _halves(logp[0])

    return (logits, emb)


# speedup vs baseline: 7.0972x; 1.0821x over previous
import functools

import jax
import jax.numpy as jnp
from jax import lax
from jax.experimental import pallas as pl
from jax.experimental.pallas import tpu as pltpu
from jax.experimental.pallas import tpu_sc as plsc

N = 10000
E = 160000
NF = 256
HC = 256
NC = 128
R = 4

NSC = 2
NTILE = 16
LANES = 16
D = 128
HALF = N // NSC
ROWS_PT = 320
ACC_ROWS = NTILE * ROWS_PT
TRASH = HALF
BASEPAD = NSC * ACC_ROWS
EP_TILE = E // NTILE
CHUNK = 80
NCHUNK = EP_TILE // CHUNK
OH_REP = 1024


def _mesh():
    return plsc.VectorSubcoreMesh(
        core_axis_name="c", subcore_axis_name="s", num_cores=NSC,
        num_subcores=NTILE)


@functools.cache
def _make_count_kernel():
    @functools.partial(
        pl.kernel,
        out_type=jax.ShapeDtypeStruct((NSC, ACC_ROWS, D), jnp.float32),
        mesh=_mesh(),
        scratch_types=[
            pltpu.VMEM((EP_TILE,), jnp.int32),
            pltpu.VMEM((EP_TILE,), jnp.int32),
            pltpu.VMEM((2, CHUNK, D), jnp.float32),
            pltpu.VMEM((2, CHUNK), jnp.int32),
            pltpu.VMEM((2, CHUNK), jnp.int32),
            pltpu.VMEM_SHARED((ACC_ROWS, D), jnp.float32),
            pltpu.SemaphoreType.DMA,
            pltpu.SemaphoreType.DMA,
            pltpu.SemaphoreType.DMA,
            pltpu.SemaphoreType.DMA,
        ],
    )
    def _cnt(oh_hbm, zeros_hbm, dst_hbm, typ_hbm, out_hbm,
             dst_v, typ_v, rows_v, gidx_v, sidx_v, acc_sh,
             sg0, sg1, ss0, ss1):
        c = lax.axis_index("c")
        t = lax.axis_index("s")
        lo = c * HALF
        r0 = t * ROWS_PT
        sg = (sg0, sg1)
        ss = (ss0, ss1)

        pltpu.sync_copy(zeros_hbm.at[pl.ds(r0, ROWS_PT)],
                        acc_sh.at[pl.ds(r0, ROWS_PT)])
        e0 = t * EP_TILE
        pltpu.sync_copy(dst_hbm.at[pl.ds(e0, EP_TILE)], dst_v)
        pltpu.sync_copy(typ_hbm.at[pl.ds(e0, EP_TILE)], typ_v)
        plsc.subcore_barrier()

        def calc_idx(k, b):
            for j in range(CHUNK // LANES):
                sl = pl.ds(j * LANES, LANES)
                esl = pl.ds(k * CHUNK + j * LANES, LANES)
                tv = typ_v[esl]
                dv = dst_v[esl]
                gidx_v[b, sl] = tv * OH_REP + (dv & (OH_REP - 1))
                li = dv - lo
                ok = (li >= 0) & (li < HALF)
                sidx_v[b, sl] = jnp.where(ok, li, TRASH)

        def step(k, b, prefetch):
            pltpu.make_async_copy(oh_hbm.at[gidx_v.at[b]], rows_v.at[b],
                                  sg[b]).wait()
            pltpu.async_copy(rows_v.at[b], acc_sh.at[sidx_v.at[b]],
                             ss[b], add=True)
            pltpu.make_async_copy(rows_v.at[b], acc_sh.at[sidx_v.at[b]],
                                  ss[b]).wait()
            if prefetch:
                @pl.when(k + 2 < NCHUNK)
                def _():
                    calc_idx(k + 2, b)
                    pltpu.async_copy(oh_hbm.at[gidx_v.at[b]],
                                     rows_v.at[b], sg[b])

        for b in range(2):
            calc_idx(b, b)
            pltpu.async_copy(oh_hbm.at[gidx_v.at[b]], rows_v.at[b], sg[b])

        def pair_body(k2, carry):
            for b in range(2):
                step(k2 * 2 + b, b, True)
            return carry
        lax.fori_loop(0, NCHUNK // 2, pair_body, 0)
        step(NCHUNK - 1, (NCHUNK - 1) % 2, False)

        plsc.subcore_barrier()
        pltpu.sync_copy(acc_sh.at[pl.ds(r0, ROWS_PT)],
                        out_hbm.at[c, pl.ds(r0, ROWS_PT)])
    return _cnt


@functools.cache
def _make_msg_kernel(npass):
    @functools.partial(
        pl.kernel,
        out_type=jax.ShapeDtypeStruct((npass, NSC, ACC_ROWS, D),
                                      jnp.float32),
        mesh=_mesh(),
        scratch_types=[
            pltpu.VMEM((EP_TILE,), jnp.int32),
            pltpu.VMEM((EP_TILE,), jnp.int32),
            pltpu.VMEM((EP_TILE,), jnp.int32),
            pltpu.VMEM((2, CHUNK, D), jnp.float32),
            pltpu.VMEM((2, CHUNK, D), jnp.float32),
            pltpu.VMEM((2, CHUNK), jnp.int32),
            pltpu.VMEM((2, CHUNK), jnp.int32),
            pltpu.VMEM((2, CHUNK), jnp.int32),
            pltpu.VMEM_SHARED((ACC_ROWS, D), jnp.float32),
            pltpu.SemaphoreType.DMA,
            pltpu.SemaphoreType.DMA,
            pltpu.SemaphoreType.DMA,
            pltpu.SemaphoreType.DMA,
            pltpu.SemaphoreType.DMA,
            pltpu.SemaphoreType.DMA,
        ],
    )
    def _msg(*args):
        bases = args[:npass]
        (hmsg_hbm, cnt_hbm, src_hbm, dst_hbm, typ_hbm, out_hbm,
         src_v, dst_v, typ_v, rows_v, crow_v, gidx_v, cidx_v, sidx_v,
         acc_sh, sgm0, sgm1, sgc0, sgc1, ss0, ss1) = args[npass:]
        c = lax.axis_index("c")
        t = lax.axis_index("s")
        lo = c * HALF
        r0 = t * ROWS_PT
        sgm = (sgm0, sgm1)
        sgc = (sgc0, sgc1)
        ss = (ss0, ss1)

        e0 = t * EP_TILE
        pltpu.sync_copy(src_hbm.at[pl.ds(e0, EP_TILE)], src_v)
        pltpu.sync_copy(dst_hbm.at[pl.ds(e0, EP_TILE)], dst_v)
        pltpu.sync_copy(typ_hbm.at[pl.ds(e0, EP_TILE)], typ_v)

        for f in range(npass):
            base_hbm = bases[f]
            pltpu.sync_copy(base_hbm.at[pl.ds(c * ACC_ROWS + r0, ROWS_PT)],
                            acc_sh.at[pl.ds(r0, ROWS_PT)])
            plsc.subcore_barrier()

            def calc_idx(k, b):
                for j in range(CHUNK // LANES):
                    sl = pl.ds(j * LANES, LANES)
                    esl = pl.ds(k * CHUNK + j * LANES, LANES)
                    sv = src_v[esl]
                    tv = typ_v[esl]
                    dv = dst_v[esl]
                    gidx_v[b, sl] = (sv * R + tv) * npass + f
                    cidx_v[b, sl] = tv * N + dv
                    li = dv - lo
                    ok = (li >= 0) & (li < HALF)
                    sidx_v[b, sl] = jnp.where(ok, li, TRASH)

            def fire_gathers(b):
                pltpu.async_copy(hmsg_hbm.at[gidx_v.at[b]], rows_v.at[b],
                                 sgm[b])
                pltpu.async_copy(cnt_hbm.at[cidx_v.at[b]], crow_v.at[b],
                                 sgc[b])

            def wait_gathers(b):
                pltpu.make_async_copy(hmsg_hbm.at[gidx_v.at[b]],
                                      rows_v.at[b], sgm[b]).wait()
                pltpu.make_async_copy(cnt_hbm.at[cidx_v.at[b]],
                                      crow_v.at[b], sgc[b]).wait()

            def wait_scatter(b):
                pltpu.make_async_copy(rows_v.at[b],
                                      acc_sh.at[sidx_v.at[b]],
                                      ss[b]).wait()

            def scale(b):
                def rbody(i, carry2):
                    bc = crow_v[b, i, pl.ds(0, LANES)]
                    for j2 in range(D // LANES):
                        sl2 = pl.ds(j2 * LANES, LANES)
                        rows_v[b, i, sl2] = rows_v[b, i, sl2] * bc
                    return carry2
                lax.fori_loop(0, CHUNK, rbody, 0)

            def step(k, b, prefetch):
                wait_gathers(b)
                scale(b)
                pltpu.async_copy(rows_v.at[b], acc_sh.at[sidx_v.at[b]],
                                 ss[b], add=True)
                wait_scatter(b)
                if prefetch:
                    @pl.when(k + 2 < NCHUNK)
                    def _():
                        calc_idx(k + 2, b)
                        fire_gathers(b)

            for b in range(2):
                calc_idx(b, b)
                fire_gathers(b)

            def pair_body(k2, carry):
                for b in range(2):
                    step(k2 * 2 + b, b, True)
                return carry
            lax.fori_loop(0, NCHUNK // 2, pair_body, 0)
            step(NCHUNK - 1, (NCHUNK - 1) % 2, False)

            plsc.subcore_barrier()
            pltpu.sync_copy(acc_sh.at[pl.ds(r0, ROWS_PT)],
                            out_hbm.at[f, c, pl.ds(r0, ROWS_PT)])
            if f + 1 < npass:
                plsc.subcore_barrier()
    return _msg


def _tc_layer(x, root, wcat, b2d, relu, d):
    bm = 1000
    grid = (N // bm,)

    def body(x_ref, root_ref, wcat_ref, b_ref, base_ref, hmsg_ref):
        xb = x_ref[...]
        if relu:
            xb = jnp.maximum(xb, 0.0)
        base_ref[...] = (
            jnp.dot(xb, root_ref[...], preferred_element_type=jnp.float32)
            + b_ref[...])
        hmsg_ref[...] = jnp.dot(
            xb, wcat_ref[...], preferred_element_type=jnp.float32)

    k = x.shape[1]
    return pl.pallas_call(
        body,
        grid=grid,
        in_specs=[
            pl.BlockSpec((bm, k), lambda i: (i, 0)),
            pl.BlockSpec((k, d), lambda i: (0, 0)),
            pl.BlockSpec((k, R * d), lambda i: (0, 0)),
            pl.BlockSpec((1, d), lambda i: (0, 0)),
        ],
        out_specs=[
            pl.BlockSpec((bm, d), lambda i: (i, 0)),
            pl.BlockSpec((bm, R * d), lambda i: (i, 0)),
        ],
        out_shape=[
            jax.ShapeDtypeStruct((N, d), jnp.float32),
            jax.ShapeDtypeStruct((N, R * d), jnp.float32),
        ],
    )(x, root, wcat, b2d)


def _halves(p):
    return jnp.concatenate([p[0, :HALF], p[1, :HALF]], axis=0)


def _pad_rows(a):
    return jnp.concatenate(
        [jnp.pad(a[:HALF], ((0, ACC_ROWS - HALF), (0, 0))),
         jnp.pad(a[HALF:], ((0, ACC_ROWS - HALF), (0, 0)))], axis=0)


def kernel(x, edge_index, edge_type, w1, root1, b1, w2, root2, b2):
    src = edge_index[0]
    dst = edge_index[1]
    typ = edge_type

    oh1 = jnp.pad(jnp.repeat(jnp.eye(R, dtype=jnp.float32), LANES, axis=1),
                  ((0, 0), (0, D - R * LANES)))
    oh_tab = jnp.broadcast_to(
        oh1[:, None, :], (R, OH_REP, D)).reshape(R * OH_REP, D)
    zeros = jnp.zeros((ACC_ROWS, D), jnp.float32)
    cntq = _make_count_kernel()(oh_tab, zeros, dst, typ)
    cvals = _halves(cntq)[:, 0:R * LANES:LANES]
    scal = 1.0 / jnp.maximum(cvals, 1.0)
    cnt = jnp.broadcast_to(
        scal.T.reshape(R * N, 1), (R * N, D))

    wcat1 = jnp.transpose(w1, (1, 0, 2)).reshape(NF, R * HC)
    wcat2 = jnp.transpose(w2, (1, 0, 2)).reshape(HC, R * NC)

    base1, hmsg1 = _tc_layer(x, root1, wcat1, b1.reshape(1, HC), False, HC)
    h1v = hmsg1.reshape(N * R * 2, D)
    embp = _make_msg_kernel(2)(
        _pad_rows(base1[:, :D]), _pad_rows(base1[:, D:]), h1v,
        cnt, src, dst, typ)
    emb = jnp.concatenate([_halves(embp[0]), _halves(embp[1])], axis=1)

    base2, hmsg2 = _tc_layer(emb, root2, wcat2, b2.reshape(1, NC), True, NC)
    logp = _make_msg_kernel(1)(
        _pad_rows(base2), hmsg2.reshape(N * R, NC), cnt, src, dst, typ)
    logits = _halves(logp[0])

    return (logits, emb)
